# Initial kernel scaffold; baseline (speedup 1.0000x reference)
import jax
import jax.numpy as jnp
from jax.experimental import pallas as pl


def kernel(y_features, edge_index, W1, gamma, beta):
    raise NotImplementedError("write your pallas kernel here")


# speedup vs baseline: 4.8573x; 4.8573x over previous
import functools

import jax
import jax.numpy as jnp
from jax import lax
from jax.experimental import pallas as pl
from jax.experimental.pallas import tpu as pltpu
from jax.experimental.pallas import tpu_sc as plsc

_EPS = 1e-5

_C = 128
_NW = 32
_NSUB = 16


def _support_body(y_ref, w_ref, out_ref):
    out_ref[...] = jnp.dot(y_ref[...], w_ref[...],
                           preferred_element_type=jnp.float32)


def _support_matmul(y, w):
    n, h_in = y.shape
    h_out = w.shape[1]
    bm = 2000
    return pl.pallas_call(
        _support_body,
        grid=(n // bm,),
        in_specs=[pl.BlockSpec((bm, h_in), lambda i: (i, 0)),
                  pl.BlockSpec((h_in, h_out), lambda i: (0, 0))],
        out_specs=pl.BlockSpec((bm, h_out), lambda i: (i, 0)),
        out_shape=jax.ShapeDtypeStruct((n, h_out), jnp.float32),
    )(y, w)


def _spmm_sc(support, src, dst, zeros):
    n, h = support.shape
    e = src.shape[0]
    nchunks = e // _C
    nloop = (nchunks + _NW - 1) // _NW
    rps = (n // _NSUB) // 8 * 8
    rps_last = n - (_NSUB - 1) * rps
    mesh = plsc.VectorSubcoreMesh(core_axis_name="c", subcore_axis_name="s")

    @functools.partial(
        pl.kernel, mesh=mesh,
        out_type=jax.ShapeDtypeStruct((2, n, h), jnp.float32),
        scratch_types=[
            pltpu.VMEM((_C,), jnp.int32),
            pltpu.VMEM((_C,), jnp.int32),
            pltpu.VMEM((_C, h), jnp.float32),
            pltpu.VMEM_SHARED((n, h), jnp.float32),
            pltpu.SemaphoreType.DMA,
        ],
    )
    def spmm(support_hbm, src_hbm, dst_hbm, zeros_hbm, out_hbm,
             src_v, dst_v, rows_v, acc_sh, sem):
        cid = lax.axis_index("c")
        sid = lax.axis_index("s")
        wid = sid * 2 + cid
        @pl.when(sid < _NSUB - 1)
        def _():
            pltpu.sync_copy(zeros_hbm.at[pl.ds(sid * rps, rps)],
                            acc_sh.at[pl.ds(sid * rps, rps)])

        @pl.when(sid == _NSUB - 1)
        def _():
            pltpu.sync_copy(zeros_hbm.at[pl.ds((_NSUB - 1) * rps, rps_last)],
                            acc_sh.at[pl.ds((_NSUB - 1) * rps, rps_last)])

        plsc.subcore_barrier()

        @pl.loop(0, nloop)
        def _(i):
            g = wid + i * _NW

            @pl.when(g < nchunks)
            def _():
                base = g * _C
                pltpu.sync_copy(src_hbm.at[pl.ds(base, _C)], src_v)
                pltpu.sync_copy(dst_hbm.at[pl.ds(base, _C)], dst_v)
                pltpu.async_copy(support_hbm.at[src_v], rows_v, sem).wait()
                pltpu.sync_copy(rows_v, acc_sh.at[dst_v], add=True)

        plsc.subcore_barrier()

        @pl.when(sid < _NSUB - 1)
        def _():
            pltpu.sync_copy(acc_sh.at[pl.ds(sid * rps, rps)],
                            out_hbm.at[cid].at[pl.ds(sid * rps, rps)])

        @pl.when(sid == _NSUB - 1)
        def _():
            pltpu.sync_copy(
                acc_sh.at[pl.ds((_NSUB - 1) * rps, rps_last)],
                out_hbm.at[cid].at[pl.ds((_NSUB - 1) * rps, rps_last)])

    return spmm(support, src, dst, zeros)


def _bn_body(p_ref, g_ref, b_ref, hn_ref):
    h = jnp.maximum(p_ref[0] + p_ref[1], 0.0)
    mean = jnp.mean(h, axis=0, keepdims=True)
    var = jnp.mean(h * h, axis=0, keepdims=True) - mean * mean
    inv = lax.rsqrt(var + _EPS)
    hn_ref[...] = (h - mean) * (inv * g_ref[...]) + b_ref[...]


def _bn(parts, gamma, beta):
    _, n, h = parts.shape
    return pl.pallas_call(
        _bn_body,
        in_specs=[pl.BlockSpec((2, n, h), lambda: (0, 0, 0)),
                  pl.BlockSpec((1, h), lambda: (0, 0)),
                  pl.BlockSpec((1, h), lambda: (0, 0))],
        out_specs=pl.BlockSpec((n, h), lambda: (0, 0)),
        out_shape=jax.ShapeDtypeStruct((n, h), jnp.float32),
    )(parts, gamma, beta)


def _gram_body(a_ref, b_ref, out_ref):
    out_ref[...] = lax.dot_general(
        a_ref[...], b_ref[...], (((1,), (1,)), ((), ())),
        preferred_element_type=jnp.float32)


def _gram(hn):
    n, h = hn.shape
    bm = 2048
    g = pl.cdiv(n, bm)
    return pl.pallas_call(
        _gram_body,
        grid=(g, g),
        in_specs=[pl.BlockSpec((bm, h), lambda i, j: (i, 0)),
                  pl.BlockSpec((bm, h), lambda i, j: (j, 0))],
        out_specs=pl.BlockSpec((bm, bm), lambda i, j: (i, j)),
        out_shape=jax.ShapeDtypeStruct((n, n), jnp.float32),
    )(hn, hn)


def kernel(y_features, edge_index, W1, gamma, beta):
    n, h = y_features.shape
    support = _support_matmul(y_features, W1)
    zeros = jnp.zeros((n, W1.shape[1]), jnp.float32)
    parts = _spmm_sc(support, edge_index[0], edge_index[1], zeros)
    hn = _bn(parts, gamma.reshape(1, h), beta.reshape(1, h))
    return _gram(hn)
